# Initial kernel scaffold; baseline (speedup 1.0000x reference)
import jax
import jax.numpy as jnp
from jax.experimental import pallas as pl


def kernel(x, edge_index, Ws1, Wn1, b1, Ws2, Wn2, b2, Wp, bp):
    raise NotImplementedError("write your pallas kernel here")


# speedup vs baseline: 4.7977x; 4.7977x over previous
import functools

import jax
import jax.numpy as jnp
from jax import lax
from jax.experimental import pallas as pl
from jax.experimental.pallas import tpu as pltpu
from jax.experimental.pallas import tpu_sc as plsc

_NC = 2
_NS = 16
_NW = _NC * _NS
_LANES = 16

_N = 10000
_E = 320000
_EPW = _E // _NW
_CHUNK = 80
_NCHUNK = _EPW // _CHUNK
_TROWS = 624
_TAIL = _N - _NS * _TROWS
_ZROWS = 104
_DROWS = 80


def _mesh():
    return plsc.VectorSubcoreMesh(core_axis_name="c", subcore_axis_name="s")


_SC_PARAMS = pltpu.CompilerParams(needs_layout_passes=False)


def _sc_agg_body(with_deg, feat_hbm, src_hbm, dst_hbm, zrows_hbm, *rest):
    if with_deg:
        (sum_hbm, deg_hbm, sidx, didx, rows, degloc, rowidx, accx, accd,
         sem) = rest
    else:
        sum_hbm, sidx, didx, rows, accx, sem = rest

    cid = lax.axis_index("c")
    sid = lax.axis_index("s")
    wid = cid * _NS + sid
    i16 = lax.iota(jnp.int32, _LANES)
    zf = jnp.zeros((_LANES,), jnp.float32)
    one16 = jnp.full((_LANES,), 1.0, jnp.float32)

    pltpu.sync_copy(zrows_hbm, accx.at[pl.ds(sid * _TROWS, _TROWS)])
    if with_deg:
        def zdl(r, carry):
            for j in range(128 // _LANES):
                degloc[r, pl.ds(j * _LANES, _LANES)] = zf
            return carry
        lax.fori_loop(0, _DROWS, zdl, 0)
        for j in range(_DROWS // _LANES):
            rowidx[pl.ds(j * _LANES, _LANES)] = i16 + j * _LANES

    @pl.when(sid == 0)
    def _zero_tail():
        tbase = _NS * _TROWS
        pltpu.sync_copy(zrows_hbm.at[pl.ds(0, _TAIL)],
                        accx.at[pl.ds(tbase, _TAIL)])
        if with_deg:
            pltpu.sync_copy(zrows_hbm.at[pl.ds(0, _DROWS)], accd)
    plsc.subcore_barrier()

    def chunk(c, carry):
        base = wid * _EPW + c * _CHUNK
        pltpu.sync_copy(src_hbm.at[pl.ds(base, _CHUNK)], sidx)
        pltpu.sync_copy(dst_hbm.at[pl.ds(base, _CHUNK)], didx)
        pltpu.async_copy(feat_hbm.at[sidx], rows, sem).wait()
        pltpu.sync_copy(rows, accx.at[didx], add=True)
        if with_deg:
            for g in range(_CHUNK // _LANES):
                d = didx[pl.ds(g * _LANES, _LANES)]
                plsc.addupdate_scatter(degloc, [d >> 7, d & 127], one16)
        return carry
    lax.fori_loop(0, _NCHUNK, chunk, 0)

    if with_deg:
        pltpu.sync_copy(degloc, accd.at[rowidx], add=True)
    plsc.subcore_barrier()

    rbase = sid * _TROWS
    pltpu.sync_copy(accx.at[pl.ds(rbase, _TROWS)],
                    sum_hbm.at[cid, pl.ds(rbase, _TROWS)])
    if with_deg:
        @pl.when(sid < _DROWS // 8)
        def _copy_deg():
            pltpu.sync_copy(accd.at[pl.ds(sid * 8, 8)],
                            deg_hbm.at[cid, pl.ds(sid * 8, 8)])

    @pl.when(sid == 0)
    def _copy_tail():
        tbase = _NS * _TROWS
        pltpu.sync_copy(accx.at[pl.ds(tbase, _TAIL)],
                        sum_hbm.at[cid, pl.ds(tbase, _TAIL)])


def _make_sc_agg(with_deg):
    out_type = [jax.ShapeDtypeStruct((_NC, _N, 128), jnp.float32)]
    scratch = [
        pltpu.VMEM((_CHUNK,), jnp.int32),
        pltpu.VMEM((_CHUNK,), jnp.int32),
        pltpu.VMEM((_CHUNK, 128), jnp.float32),
    ]
    if with_deg:
        out_type.append(jax.ShapeDtypeStruct((_NC, _DROWS, 128), jnp.float32))
        scratch.append(pltpu.VMEM((_DROWS, 128), jnp.float32))
        scratch.append(pltpu.VMEM((_DROWS,), jnp.int32))
    scratch.append(pltpu.VMEM_SHARED((_N, 128), jnp.float32))
    if with_deg:
        scratch.append(pltpu.VMEM_SHARED((_DROWS, 128), jnp.float32))
    scratch.append(pltpu.SemaphoreType.DMA)
    return functools.partial(
        pl.kernel,
        functools.partial(_sc_agg_body, with_deg),
        mesh=_mesh(),
        out_type=out_type,
        scratch_types=scratch,
        compiler_params=_SC_PARAMS,
    )()


def _sc_score_body(sa_hbm, sb_hbm, src_hbm, dst_hbm, out_hbm,
                   sav, sbv, sidx, didx, outv, sem):
    cid = lax.axis_index("c")
    sid = lax.axis_index("s")
    wid = cid * _NS + sid
    pltpu.sync_copy(sa_hbm, sav)
    pltpu.sync_copy(sb_hbm, sbv)
    i16 = lax.iota(jnp.int32, _LANES)

    def chunk(c, carry):
        base = wid * _EPW + c * _CHUNK
        pltpu.sync_copy(src_hbm.at[pl.ds(base, _CHUNK)], sidx)
        pltpu.sync_copy(dst_hbm.at[pl.ds(base, _CHUNK)], didx)
        for g in range(_CHUNK // _LANES):
            s2 = sidx[pl.ds(g * _LANES, _LANES)] * 2
            d2 = didx[pl.ds(g * _LANES, _LANES)] * 2
            a0 = plsc.load_gather(sav, [s2])
            a1 = plsc.load_gather(sav, [s2 + 1])
            b0 = plsc.load_gather(sbv, [d2])
            b1 = plsc.load_gather(sbv, [d2 + 1])
            p = (i16 + g * _LANES) * 2
            plsc.store_scatter(outv, [p], a0 + b0)
            plsc.store_scatter(outv, [p + 1], a1 + b1)
        pltpu.sync_copy(outv, out_hbm.at[pl.ds(base * 2, _CHUNK * 2)])
        return carry
    lax.fori_loop(0, _NCHUNK, chunk, 0)


def _sc_score(sa_flat, sb_flat, src, dst):
    return pl.kernel(
        _sc_score_body,
        mesh=_mesh(),
        out_type=jax.ShapeDtypeStruct((2 * _E,), jnp.float32),
        scratch_types=[
            pltpu.VMEM((2 * _N,), jnp.float32),
            pltpu.VMEM((2 * _N,), jnp.float32),
            pltpu.VMEM((_CHUNK,), jnp.int32),
            pltpu.VMEM((_CHUNK,), jnp.int32),
            pltpu.VMEM((2 * _CHUNK,), jnp.float32),
            pltpu.SemaphoreType.DMA,
        ],
        compiler_params=_SC_PARAMS,
    )(sa_flat, sb_flat, src, dst)


_BN = 1000


def _tc_layer1_body(x_ref, sx0, sx1, d0, d1, ws1, wn1, b1, wn2,
                    h1_ref, t2_ref, rec_ref):
    deg = d0[...] + d1[...]
    rec = 1.0 / jnp.maximum(deg, 1.0)
    agg = (sx0[...] + sx1[...]) * rec
    h1 = jnp.dot(x_ref[...], ws1[...], preferred_element_type=jnp.float32)
    h1 = h1 + jnp.dot(agg, wn1[...], preferred_element_type=jnp.float32)
    h1 = jnp.maximum(h1 + b1[...], 0.0)
    h1_ref[...] = h1
    t2_ref[...] = jnp.dot(h1, wn2[...], preferred_element_type=jnp.float32)
    rec_ref[...] = rec


def _tc_layer1(x, sx0, sx1, d0, d1, Ws1, Wn1, b1, Wn2):
    grid = (_N // _BN,)
    row = lambda i: (i, 0)
    full = lambda i: (0, 0)
    return pl.pallas_call(
        _tc_layer1_body,
        grid=grid,
        in_specs=[
            pl.BlockSpec((_BN, 128), row),
            pl.BlockSpec((_BN, 128), row),
            pl.BlockSpec((_BN, 128), row),
            pl.BlockSpec((_BN, 1), row),
            pl.BlockSpec((_BN, 1), row),
            pl.BlockSpec((128, 256), full),
            pl.BlockSpec((128, 256), full),
            pl.BlockSpec((1, 256), full),
            pl.BlockSpec((256, 128), full),
        ],
        out_specs=[
            pl.BlockSpec((_BN, 256), row),
            pl.BlockSpec((_BN, 128), row),
            pl.BlockSpec((_BN, 1), row),
        ],
        out_shape=[
            jax.ShapeDtypeStruct((_N, 256), jnp.float32),
            jax.ShapeDtypeStruct((_N, 128), jnp.float32),
            jax.ShapeDtypeStruct((_N, 1), jnp.float32),
        ],
    )(x, sx0, sx1, d0, d1, Ws1, Wn1, b1, Wn2)


def _tc_layer2_body(h1_ref, sy0, sy1, rec_ref, ws2, b2, wpa, wpb, bp,
                    sa_ref, sb_ref):
    h2 = jnp.dot(h1_ref[...], ws2[...], preferred_element_type=jnp.float32)
    h2 = h2 + (sy0[...] + sy1[...]) * rec_ref[...] + b2[...]
    sa_ref[...] = jnp.dot(h2, wpa[...], preferred_element_type=jnp.float32) + bp[...]
    sb_ref[...] = jnp.dot(h2, wpb[...], preferred_element_type=jnp.float32)


def _tc_layer2(h1, sy0, sy1, rec, Ws2, b2, WpA, WpB, bp):
    grid = (_N // _BN,)
    row = lambda i: (i, 0)
    full = lambda i: (0, 0)
    return pl.pallas_call(
        _tc_layer2_body,
        grid=grid,
        in_specs=[
            pl.BlockSpec((_BN, 256), row),
            pl.BlockSpec((_BN, 128), row),
            pl.BlockSpec((_BN, 128), row),
            pl.BlockSpec((_BN, 1), row),
            pl.BlockSpec((256, 128), full),
            pl.BlockSpec((1, 128), full),
            pl.BlockSpec((128, 2), full),
            pl.BlockSpec((128, 2), full),
            pl.BlockSpec((1, 2), full),
        ],
        out_specs=[
            pl.BlockSpec((_BN, 2), row),
            pl.BlockSpec((_BN, 2), row),
        ],
        out_shape=[
            jax.ShapeDtypeStruct((_N, 2), jnp.float32),
            jax.ShapeDtypeStruct((_N, 2), jnp.float32),
        ],
    )(h1, sy0, sy1, rec, Ws2, b2, WpA, WpB, bp)


def kernel(x, edge_index, Ws1, Wn1, b1, Ws2, Wn2, b2, Wp, bp):
    src = edge_index[0]
    dst = edge_index[1]
    zrows = jnp.zeros((_TROWS, 128), jnp.float32)

    sum_x, deg_w = _make_sc_agg(True)(x, src, dst, zrows)
    d0 = deg_w[0].reshape(-1)[:_N].reshape(_N, 1)
    d1 = deg_w[1].reshape(-1)[:_N].reshape(_N, 1)
    h1, t2, rec = _tc_layer1(x, sum_x[0], sum_x[1], d0, d1,
                             Ws1, Wn1, b1.reshape(1, 256), Wn2)
    (sum_y,) = _make_sc_agg(False)(t2, src, dst, zrows)
    sa, sb = _tc_layer2(h1, sum_y[0], sum_y[1], rec, Ws2,
                        b2.reshape(1, 128), Wp[:128], Wp[128:],
                        bp.reshape(1, 2))
    return _sc_score(sa.reshape(-1), sb.reshape(-1), src, dst).reshape(_E, 2)


# speedup vs baseline: 9.2911x; 1.9366x over previous
import functools

import jax
import jax.numpy as jnp
from jax import lax
from jax.experimental import pallas as pl
from jax.experimental.pallas import tpu as pltpu
from jax.experimental.pallas import tpu_sc as plsc

_NC = 2
_NS = 16
_NW = _NC * _NS
_LANES = 16

_N = 10000
_E = 320000
_EPW = _E // _NW
_CHUNK = 80
_NCHUNK = _EPW // _CHUNK
_TROWS = 624
_TAIL = _N - _NS * _TROWS
_ZROWS = 104
_DROWS = 80
_NBUF = 5
_LOOK = 3


def _mesh():
    return plsc.VectorSubcoreMesh(core_axis_name="c", subcore_axis_name="s")


_SC_PARAMS = pltpu.CompilerParams(needs_layout_passes=False)


def _sc_agg_body(with_deg, feat_hbm, src_hbm, dst_hbm, zrows_hbm, *rest):
    if with_deg:
        sum_hbm, deg_hbm, degloc, accx, *bufs = rest
    else:
        sum_hbm, accx, *bufs = rest
    rows = bufs[:3]
    sidxb = bufs[3:7]
    didxb = bufs[7:11]
    gsem = bufs[11:14]
    isem = bufs[14:18]
    ssem = bufs[18:20]

    cid = lax.axis_index("c")
    sid = lax.axis_index("s")
    wid = cid * _NS + sid
    zf = jnp.zeros((_LANES,), jnp.float32)
    one16 = jnp.full((_LANES,), 1.0, jnp.float32)
    ebase = wid * _EPW

    def issue_idx(c, j):
        pltpu.async_copy(src_hbm.at[pl.ds(ebase + c * _CHUNK, _CHUNK)],
                         sidxb[j], isem[j])
        pltpu.async_copy(dst_hbm.at[pl.ds(ebase + c * _CHUNK, _CHUNK)],
                         didxb[j], isem[j])

    def wait_idx(c, j):
        pltpu.make_async_copy(src_hbm.at[pl.ds(ebase + c * _CHUNK, _CHUNK)],
                              sidxb[j], isem[j]).wait()
        pltpu.make_async_copy(dst_hbm.at[pl.ds(ebase + c * _CHUNK, _CHUNK)],
                              didxb[j], isem[j]).wait()

    issue_idx(0, 0)
    issue_idx(1, 1)
    pltpu.sync_copy(zrows_hbm, accx.at[pl.ds(sid * _TROWS, _TROWS)])
    if with_deg:
        def zdl(r, carry):
            for j in range(128 // _LANES):
                degloc[r, pl.ds(j * _LANES, _LANES)] = zf
            return carry
        lax.fori_loop(0, _DROWS, zdl, 0)

    @pl.when(sid == 0)
    def _zero_tail():
        pltpu.sync_copy(zrows_hbm.at[pl.ds(0, _TAIL)],
                        accx.at[pl.ds(_NS * _TROWS, _TAIL)])
    plsc.subcore_barrier()
    wait_idx(0, 0)
    pltpu.async_copy(feat_hbm.at[sidxb[0]], rows[0], gsem[0])

    def slot(c, k):
        @pl.when(c >= 2)
        def _wait_scat():
            pltpu.make_async_copy(rows[(k - 2) % 3],
                                  accx.at[didxb[(k - 2) % 4]],
                                  ssem[k % 2]).wait()

        @pl.when(c + 2 < _NCHUNK)
        def _pref_idx():
            issue_idx(c + 2, (k + 2) % 4)

        @pl.when(c + 1 < _NCHUNK)
        def _next_gather():
            wait_idx(c + 1, (k + 1) % 4)
            pltpu.async_copy(feat_hbm.at[sidxb[(k + 1) % 4]],
                             rows[(k + 1) % 3], gsem[(k + 1) % 3])
        pltpu.make_async_copy(feat_hbm.at[sidxb[k % 4]], rows[k % 3],
                              gsem[k % 3]).wait()
        if with_deg:
            for g in range(_CHUNK // _LANES):
                d = didxb[k % 4][pl.ds(g * _LANES, _LANES)]
                plsc.addupdate_scatter(degloc, [d >> 7, d & 127], one16)
        pltpu.async_copy(rows[k % 3], accx.at[didxb[k % 4]], ssem[k % 2],
                         add=True)

    def outer(o, carry):
        for k in range(12):
            slot(o * 12 + k, k)
        return carry
    lax.fori_loop(0, _NCHUNK // 12, outer, 0)
    for c in range((_NCHUNK // 12) * 12, _NCHUNK):
        slot(c, c)

    for c in (_NCHUNK - 2, _NCHUNK - 1):
        pltpu.make_async_copy(rows[c % 3], accx.at[didxb[c % 4]],
                              ssem[c % 2]).wait()
    if with_deg:
        pltpu.sync_copy(degloc, deg_hbm.at[wid])
    plsc.subcore_barrier()

    rbase = sid * _TROWS
    pltpu.sync_copy(accx.at[pl.ds(rbase, _TROWS)],
                    sum_hbm.at[cid, pl.ds(rbase, _TROWS)])

    @pl.when(sid == 0)
    def _copy_tail():
        tbase = _NS * _TROWS
        pltpu.sync_copy(accx.at[pl.ds(tbase, _TAIL)],
                        sum_hbm.at[cid, pl.ds(tbase, _TAIL)])


def _make_sc_agg(with_deg):
    out_type = [jax.ShapeDtypeStruct((_NC, _N, 128), jnp.float32)]
    scratch = []
    if with_deg:
        out_type.append(jax.ShapeDtypeStruct((_NW, _DROWS, 128), jnp.float32))
        scratch.append(pltpu.VMEM((_DROWS, 128), jnp.float32))
    scratch.append(pltpu.VMEM_SHARED((_N, 128), jnp.float32))
    for _ in range(3):
        scratch.append(pltpu.VMEM((_CHUNK, 128), jnp.float32))
    for _ in range(8):
        scratch.append(pltpu.VMEM((_CHUNK,), jnp.int32))
    for _ in range(9):
        scratch.append(pltpu.SemaphoreType.DMA)
    return functools.partial(
        pl.kernel,
        functools.partial(_sc_agg_body, with_deg),
        mesh=_mesh(),
        out_type=out_type,
        scratch_types=scratch,
        compiler_params=_SC_PARAMS,
    )()


def _sc_score_body(sa_hbm, sb_hbm, src_hbm, dst_hbm, out_hbm,
                   sav, sbv, sidx2, didx2, outv):
    cid = lax.axis_index("c")
    sid = lax.axis_index("s")
    wid = cid * _NS + sid
    pltpu.sync_copy(sa_hbm, sav)
    pltpu.sync_copy(sb_hbm, sbv)
    pltpu.sync_copy(src_hbm.at[wid], sidx2)
    pltpu.sync_copy(dst_hbm.at[wid], didx2)
    i16 = lax.iota(jnp.int32, _LANES)

    def chunk(c, carry):
        for g in range(_CHUNK // _LANES):
            s2 = sidx2[c, pl.ds(g * _LANES, _LANES)] * 2
            d2 = didx2[c, pl.ds(g * _LANES, _LANES)] * 2
            a0 = plsc.load_gather(sav, [s2])
            a1 = plsc.load_gather(sav, [s2 + 1])
            b0 = plsc.load_gather(sbv, [d2])
            b1 = plsc.load_gather(sbv, [d2 + 1])
            p = (i16 + c * _CHUNK + g * _LANES) * 2
            plsc.store_scatter(outv, [p], a0 + b0)
            plsc.store_scatter(outv, [p + 1], a1 + b1)
        return carry
    lax.fori_loop(0, _NCHUNK, chunk, 0)
    pltpu.sync_copy(outv, out_hbm.at[pl.ds(wid * 2 * _EPW, 2 * _EPW)])


def _sc_score(sa_flat, sb_flat, src3, dst3):
    return pl.kernel(
        _sc_score_body,
        mesh=_mesh(),
        out_type=jax.ShapeDtypeStruct((2 * _E,), jnp.float32),
        scratch_types=[
            pltpu.VMEM((2 * _N,), jnp.float32),
            pltpu.VMEM((2 * _N,), jnp.float32),
            pltpu.VMEM((_NCHUNK, _CHUNK), jnp.int32),
            pltpu.VMEM((_NCHUNK, _CHUNK), jnp.int32),
            pltpu.VMEM((2 * _EPW,), jnp.float32),
        ],
        compiler_params=_SC_PARAMS,
    )(sa_flat, sb_flat, src3, dst3)


_BN = 1000


def _tc_layer1_body(x_ref, sx0, sx1, dp, ws1, wn1, b1, wn2,
                    h1_ref, t2_ref, rec_ref):
    deg = jnp.sum(dp[...], axis=1, keepdims=True)
    rec = 1.0 / jnp.maximum(deg, 1.0)
    agg = (sx0[...] + sx1[...]) * rec
    h1 = jnp.dot(x_ref[...], ws1[...], preferred_element_type=jnp.float32)
    h1 = h1 + jnp.dot(agg, wn1[...], preferred_element_type=jnp.float32)
    h1 = jnp.maximum(h1 + b1[...], 0.0)
    h1_ref[...] = h1
    t2_ref[...] = jnp.dot(h1, wn2[...], preferred_element_type=jnp.float32)
    rec_ref[...] = rec


def _tc_layer1(x, sx0, sx1, dp, Ws1, Wn1, b1, Wn2):
    grid = (_N // _BN,)
    row = lambda i: (i, 0)
    full = lambda i: (0, 0)
    return pl.pallas_call(
        _tc_layer1_body,
        grid=grid,
        in_specs=[
            pl.BlockSpec((_BN, 128), row),
            pl.BlockSpec((_BN, 128), row),
            pl.BlockSpec((_BN, 128), row),
            pl.BlockSpec((_BN, _NW), row),
            pl.BlockSpec((128, 256), full),
            pl.BlockSpec((128, 256), full),
            pl.BlockSpec((1, 256), full),
            pl.BlockSpec((256, 128), full),
        ],
        out_specs=[
            pl.BlockSpec((_BN, 256), row),
            pl.BlockSpec((_BN, 128), row),
            pl.BlockSpec((_BN, 1), row),
        ],
        out_shape=[
            jax.ShapeDtypeStruct((_N, 256), jnp.float32),
            jax.ShapeDtypeStruct((_N, 128), jnp.float32),
            jax.ShapeDtypeStruct((_N, 1), jnp.float32),
        ],
    )(x, sx0, sx1, dp, Ws1, Wn1, b1, Wn2)


def _tc_layer2_body(h1_ref, sy0, sy1, rec_ref, ws2, b2, wpa, wpb, bp,
                    sa_ref, sb_ref):
    h2 = jnp.dot(h1_ref[...], ws2[...], preferred_element_type=jnp.float32)
    h2 = h2 + (sy0[...] + sy1[...]) * rec_ref[...] + b2[...]
    sa_ref[...] = jnp.dot(h2, wpa[...], preferred_element_type=jnp.float32) + bp[...]
    sb_ref[...] = jnp.dot(h2, wpb[...], preferred_element_type=jnp.float32)


def _tc_layer2(h1, sy0, sy1, rec, Ws2, b2, WpA, WpB, bp):
    grid = (_N // _BN,)
    row = lambda i: (i, 0)
    full = lambda i: (0, 0)
    return pl.pallas_call(
        _tc_layer2_body,
        grid=grid,
        in_specs=[
            pl.BlockSpec((_BN, 256), row),
            pl.BlockSpec((_BN, 128), row),
            pl.BlockSpec((_BN, 128), row),
            pl.BlockSpec((_BN, 1), row),
            pl.BlockSpec((256, 128), full),
            pl.BlockSpec((1, 128), full),
            pl.BlockSpec((128, 2), full),
            pl.BlockSpec((128, 2), full),
            pl.BlockSpec((1, 2), full),
        ],
        out_specs=[
            pl.BlockSpec((_BN, 2), row),
            pl.BlockSpec((_BN, 2), row),
        ],
        out_shape=[
            jax.ShapeDtypeStruct((_N, 2), jnp.float32),
            jax.ShapeDtypeStruct((_N, 2), jnp.float32),
        ],
    )(h1, sy0, sy1, rec, Ws2, b2, WpA, WpB, bp)


def kernel(x, edge_index, Ws1, Wn1, b1, Ws2, Wn2, b2, Wp, bp):
    src = edge_index[0]
    dst = edge_index[1]
    src3 = src.reshape(_NW, _NCHUNK, _CHUNK)
    dst3 = dst.reshape(_NW, _NCHUNK, _CHUNK)
    zrows = jnp.zeros((_TROWS, 128), jnp.float32)

    sum_x, deg_w = _make_sc_agg(True)(x, src, dst, zrows)
    degp = deg_w.reshape(_NW, _DROWS * 128)[:, :_N].T
    h1, t2, rec = _tc_layer1(x, sum_x[0], sum_x[1], degp,
                             Ws1, Wn1, b1.reshape(1, 256), Wn2)
    (sum_y,) = _make_sc_agg(False)(t2, src, dst, zrows)
    sa, sb = _tc_layer2(h1, sum_y[0], sum_y[1], rec, Ws2,
                        b2.reshape(1, 128), Wp[:128], Wp[128:],
                        bp.reshape(1, 2))
    return _sc_score(sa.reshape(-1), sb.reshape(-1), src3, dst3).reshape(_E, 2)
